# baseline (speedup 1.0000x reference)
import jax
import jax.numpy as jnp
from jax.experimental import pallas as pl
from jax.experimental.pallas import tpu as pltpu

_B, _P, _C = 64, 8732, 21
_BB = 8
_NBLK = _B // _BB
_EPS = 1e-7


def _body(conf_ref, shuf_ref, interp_ref, out_ref, acc_ref):
    j = pl.program_id(0)
    first = j == 0
    last = j == _NBLK - 1

    @pl.when(first)
    def _init():
        acc_ref[0] = 0.0
        acc_ref[1] = 0.0
        acc_ref[2] = 0.0
        acc_ref[3] = 0.0

    x0 = conf_ref[0]
    s0 = shuf_ref[0]
    mx = x0
    ms = s0
    ri0 = 1.0 / (interp_ref[0] + _EPS)
    t0 = x0 + _EPS
    u0 = s0 + _EPS
    accx = t0 * jnp.log(t0 * ri0)
    accs = u0 * jnp.log(u0 * ri0)
    for c in range(1, _C):
        xc = conf_ref[c]
        sc = shuf_ref[c]
        ric = 1.0 / (interp_ref[c] + _EPS)
        mx = jnp.maximum(mx, xc)
        ms = jnp.maximum(ms, sc)
        tx = xc + _EPS
        ts = sc + _EPS
        accx = accx + tx * jnp.log(tx * ric)
        accs = accs + ts * jnp.log(ts * ric)
    lm = mx > x0
    rm = ms > s0
    olf = jnp.logical_and(lm, jnp.logical_not(rm)).astype(jnp.float32)
    orf = jnp.logical_and(rm, jnp.logical_not(lm)).astype(jnp.float32)

    acc_ref[0] += jnp.sum(accx * olf)
    acc_ref[1] += jnp.sum(olf)
    acc_ref[2] += jnp.sum(accs * orf)
    acc_ref[3] += jnp.sum(orf)

    @pl.when(last)
    def _fin():
        sl, cl, sr, cr = acc_ref[0], acc_ref[1], acc_ref[2], acc_ref[3]
        loss_l = jnp.where(cl > 0.0, sl / jnp.maximum(cl, 1.0), 0.0)
        loss_r = jnp.where(cr > 0.0, sr / jnp.maximum(cr, 1.0), 0.0)
        out_ref[0] = loss_l + loss_r


def kernel(args, lam, conf, conf_flip, loc, loc_flip, conf_shuffle,
           conf_interpolation, loc_shuffle, loc_interpolation):
    half_blk = (_B // 2) // _BB
    nblk = _NBLK
    conf_t = jnp.transpose(conf, (2, 0, 1))
    shuf_t = jnp.transpose(conf_shuffle, (2, 0, 1))
    interp_t = jnp.transpose(conf_interpolation, (2, 0, 1))
    loss = pl.pallas_call(
        _body,
        grid=(nblk,),
        in_specs=[
            pl.BlockSpec((_C, _BB, _P), lambda j: (0, j, 0)),
            pl.BlockSpec((_C, _BB, _P),
                         lambda j: (0, (j + half_blk) % nblk, 0)),
            pl.BlockSpec((_C, _BB, _P), lambda j: (0, j, 0)),
        ],
        out_specs=pl.BlockSpec(memory_space=pltpu.SMEM),
        out_shape=jax.ShapeDtypeStruct((1,), jnp.float32),
        scratch_shapes=[pltpu.SMEM((4,), jnp.float32)],
    )(conf_t, shuf_t, interp_t)
    return (jnp.zeros((1,), jnp.float32), loss[0])
